# baseline (speedup 1.0000x reference)
import functools

import jax
import jax.numpy as jnp
from jax import lax
from jax.experimental import pallas as pl
from jax.experimental.pallas import tpu as pltpu
from jax.experimental.pallas import tpu_sc as plsc

_NUM_WORKERS = 32
_CH = 128
_PAD = 128
_LANES = 16
_R = 4


def _build_gather(num_rows, d, hist, batch):
    blocks_per_w = batch // _CH // _NUM_WORKERS
    n_ch = hist * blocks_per_w
    mesh = plsc.VectorSubcoreMesh(core_axis_name="c", subcore_axis_name="s")

    @functools.partial(
        pl.kernel,
        mesh=mesh,
        out_type=jax.ShapeDtypeStruct((hist, d, batch), jnp.float32),
        scratch_types=[
            [pltpu.VMEM((_CH,), jnp.int32) for _ in range(_R)],
            [pltpu.VMEM((_CH, _PAD), jnp.float32) for _ in range(_R)],
            [pltpu.VMEM((d, _CH), jnp.float32) for _ in range(2)],
            [pltpu.SemaphoreType.DMA for _ in range(_R)],
            [pltpu.SemaphoreType.DMA for _ in range(_R)],
            [pltpu.SemaphoreType.DMA for _ in range(2)],
        ],
        compiler_params=pltpu.CompilerParams(needs_layout_passes=False),
    )
    def gather_kernel(table_hbm, zt_hbm, out_hbm, ibufs, rbufs, sbufs, isems,
                      gsems, out_sems):
        wid = lax.axis_index("s") * 2 + lax.axis_index("c")
        col0 = wid * (blocks_per_w * _CH)

        def coords(n):
            i1 = n // blocks_per_w
            i0 = col0 + (n % blocks_per_w) * _CH
            return i1, i0

        def idx_copy(n, s):
            i1, i0 = coords(n)
            return pltpu.make_async_copy(zt_hbm.at[i1, pl.ds(i0, _CH)],
                                         ibufs[s], isems[s])

        def gather_copy(s):
            return pltpu.make_async_copy(table_hbm.at[ibufs[s]], rbufs[s],
                                         gsems[s])

        def out_copy(n, t):
            i1, i0 = coords(n)
            return pltpu.make_async_copy(sbufs[t],
                                         out_hbm.at[i1, :, pl.ds(i0, _CH)],
                                         out_sems[t])

        iota = lax.iota(jnp.int32, _LANES)
        rots = [(iota + k) % _LANES for k in range(_LANES)]

        def transpose(rbuf, sbuf):
            def bbody(b, carry):
                rows = iota + b * _LANES
                for cb in range(d // _LANES):
                    for k in range(_LANES):
                        cols = rots[k] + cb * _LANES
                        v = plsc.load_gather(rbuf, [rows, cols])
                        plsc.store_scatter(sbuf, [cols, rows], v)
                return carry

            lax.fori_loop(0, _CH // _LANES, bbody, 0)

        for s in range(_R):
            idx_copy(s, s).start()
        idx_copy(0, 0).wait()
        gather_copy(0).start()
        idx_copy(1, 1).wait()
        gather_copy(1).start()

        def step(n, s, t):
            s2 = (s + 2) % _R

            @pl.when(n + 2 < n_ch)
            def _():
                idx_copy(n + 2, s2).wait()
                gather_copy(s2).start()

            @pl.when(n >= 2)
            def _():
                out_copy(n - 2, t).wait()

            gather_copy(s).wait()
            transpose(rbufs[s], sbufs[t])
            out_copy(n, t).start()

            @pl.when(n + _R < n_ch)
            def _():
                idx_copy(n + _R, s).start()

        def body(u, carry):
            n = u * _R
            for s in range(_R):
                step(n + s, s, s % 2)
            return carry

        lax.fori_loop(0, n_ch // _R, body, 0)

        out_copy(n_ch - 2, (n_ch - 2) % 2).wait()
        out_copy(n_ch - 1, (n_ch - 1) % 2).wait()

    return gather_kernel


def kernel(properties, z):
    num_rows, d = properties.shape
    batch, hist = z.shape
    table = jnp.pad(properties, ((0, 0), (0, _PAD - d)))
    zt = z.T.astype(jnp.int32)
    out_t = _build_gather(num_rows, d, hist, batch)(table, zt)
    return out_t.transpose(2, 0, 1)


# speedup vs baseline: 1.4098x; 1.4098x over previous
import functools

import jax
import jax.numpy as jnp
from jax import lax
from jax.experimental import pallas as pl
from jax.experimental.pallas import tpu as pltpu
from jax.experimental.pallas import tpu_sc as plsc

_NUM_WORKERS = 32
_CH = 128
_PAD = 128
_LANES = 16
_R = 4


def _build_gather(num_rows, d, hist, batch):
    blocks_per_w = batch // _CH // _NUM_WORKERS
    n_ch = hist * blocks_per_w
    mesh = plsc.VectorSubcoreMesh(core_axis_name="c", subcore_axis_name="s")

    @functools.partial(
        pl.kernel,
        mesh=mesh,
        out_type=jax.ShapeDtypeStruct((hist, d, batch), jnp.float32),
        scratch_types=[
            [pltpu.VMEM((_CH,), jnp.int32) for _ in range(_R)],
            [pltpu.VMEM((_CH, _PAD), jnp.float32) for _ in range(_R)],
            [pltpu.VMEM((d, _CH), jnp.float32) for _ in range(2)],
            [pltpu.SemaphoreType.DMA for _ in range(_R)],
            [pltpu.SemaphoreType.DMA for _ in range(_R)],
            [pltpu.SemaphoreType.DMA for _ in range(2)],
        ],
        compiler_params=pltpu.CompilerParams(needs_layout_passes=False),
    )
    def gather_kernel(table_hbm, zt_hbm, out_hbm, ibufs, rbufs, sbufs, isems,
                      gsems, out_sems):
        wid = lax.axis_index("s") * 2 + lax.axis_index("c")
        col0 = wid * (blocks_per_w * _CH)

        def coords(n):
            i1 = n // blocks_per_w
            i0 = col0 + (n % blocks_per_w) * _CH
            return i1, i0

        def idx_copy(n, s):
            i1, i0 = coords(n)
            return pltpu.make_async_copy(zt_hbm.at[i1, pl.ds(i0, _CH)],
                                         ibufs[s], isems[s])

        def gather_copy(s):
            return pltpu.make_async_copy(table_hbm.at[ibufs[s]], rbufs[s],
                                         gsems[s])

        def out_copy(n, t):
            i1, i0 = coords(n)
            return pltpu.make_async_copy(sbufs[t],
                                         out_hbm.at[i1, :, pl.ds(i0, _CH)],
                                         out_sems[t])

        iota = lax.iota(jnp.int32, _LANES)
        rots = [(iota + k) % _LANES for k in range(_LANES)]
        zeros = jnp.zeros((_LANES,), jnp.int32)
        lpat = [iota * _PAD + rots[k] for k in range(_LANES)]
        spat = [rots[k] * _CH + iota for k in range(_LANES)]

        def transpose(rbuf, sbuf):
            def bbody(b, carry):
                for cb in range(d // _LANES):
                    lbase = b * (_LANES * _PAD) + cb * _LANES
                    sbase = cb * (_LANES * _CH) + b * _LANES
                    for k in range(_LANES):
                        v = plsc.load_gather(rbuf, [zeros, lpat[k] + lbase])
                        plsc.store_scatter(sbuf, [zeros, spat[k] + sbase], v)
                return carry

            lax.fori_loop(0, _CH // _LANES, bbody, 0)

        for s in range(_R):
            idx_copy(s, s).start()
        idx_copy(0, 0).wait()
        gather_copy(0).start()
        idx_copy(1, 1).wait()
        gather_copy(1).start()

        def step(n, s, t):
            s2 = (s + 2) % _R

            @pl.when(n + 2 < n_ch)
            def _():
                idx_copy(n + 2, s2).wait()
                gather_copy(s2).start()

            @pl.when(n >= 2)
            def _():
                out_copy(n - 2, t).wait()

            gather_copy(s).wait()
            transpose(rbufs[s], sbufs[t])
            out_copy(n, t).start()

            @pl.when(n + _R < n_ch)
            def _():
                idx_copy(n + _R, s).start()

        def body(u, carry):
            n = u * _R
            for s in range(_R):
                step(n + s, s, s % 2)
            return carry

        lax.fori_loop(0, n_ch // _R, body, 0)

        out_copy(n_ch - 2, (n_ch - 2) % 2).wait()
        out_copy(n_ch - 1, (n_ch - 1) % 2).wait()

    return gather_kernel


def kernel(properties, z):
    num_rows, d = properties.shape
    batch, hist = z.shape
    table = jnp.pad(properties, ((0, 0), (0, _PAD - d)))
    zt = z.T.astype(jnp.int32)
    out_t = _build_gather(num_rows, d, hist, batch)(table, zt)
    return out_t.transpose(2, 0, 1)
